# baseline (device time: 43684 ns/iter reference)
import os

import jax
import jax.numpy as jnp
from jax import lax
from jax.experimental import pallas as pl
from jax.experimental.pallas import tpu as pltpu

N_DEV = 32
N_PLANES = 4
PLANE = 8


def kernel(x, w_mat):
    m_total, k_blk = x.shape
    k_total, n_dim = w_mat.shape
    m_blk = m_total // N_DEV

    def body(x_ref, w_ref, out_ref, xs_ref, abuf_ref, xg_ref,
             sendA, recvA, sendB, recvB, pready, copy_sem):
        me = lax.axis_index("i")
        g = me // PLANE
        l = lax.rem(me, PLANE)

        xs_ref[:, :, :] = (
            x_ref[:, :].astype(jnp.bfloat16).reshape(N_DEV, m_blk, k_blk)
        )

        zready = pltpu.get_barrier_semaphore()
        for dz in range(1, N_PLANES):
            gp = lax.rem(g + dz, N_PLANES)
            pl.semaphore_signal(
                zready, inc=1,
                device_id=(gp * PLANE + l,),
                device_id_type=pl.DeviceIdType.MESH,
            )
        for dl in range(1, PLANE):
            lp = lax.rem(l + dl, PLANE)
            pl.semaphore_signal(
                pready, inc=1,
                device_id=(g * PLANE + lp,),
                device_id_type=pl.DeviceIdType.MESH,
            )

        pl.semaphore_wait(zready, N_PLANES - 1)
        rdmasA = []
        for dz in range(1, N_PLANES):
            gp = lax.rem(g + dz, N_PLANES)
            rdma = pltpu.make_async_remote_copy(
                src_ref=xs_ref.at[pl.ds(gp * PLANE, PLANE)],
                dst_ref=abuf_ref.at[g],
                send_sem=sendA.at[dz - 1],
                recv_sem=recvA.at[dz - 1],
                device_id=(gp * PLANE + l,),
                device_id_type=pl.DeviceIdType.MESH,
            )
            rdma.start()
            rdmasA.append(rdma)

        local_copy = pltpu.make_async_copy(
            xs_ref.at[pl.ds(g * PLANE, PLANE)], abuf_ref.at[g], copy_sem
        )
        local_copy.start()

        accs = [None, None, None, None]
        accs[0] = jnp.dot(
            xs_ref[me].astype(jnp.float32),
            w_ref[pl.ds(me * k_blk, k_blk), :],
            preferred_element_type=jnp.float32,
        )

        for r in rdmasA:
            r.wait_recv()
        local_copy.wait()

        pl.semaphore_wait(pready, PLANE - 1)
        rdmasB = []
        for dl in range(1, PLANE):
            lp = lax.rem(l + dl, PLANE)
            rdma = pltpu.make_async_remote_copy(
                src_ref=abuf_ref.at[:, lp],
                dst_ref=xg_ref.at[:, l],
                send_sem=sendB.at[dl - 1],
                recv_sem=recvB.at[dl - 1],
                device_id=(g * PLANE + lp,),
                device_id_type=pl.DeviceIdType.MESH,
            )
            rdma.start()
            rdmasB.append(rdma)

        for dz in range(1, N_PLANES):
            gs = lax.rem(g + (N_PLANES - dz), N_PLANES)
            p = jnp.dot(
                abuf_ref[gs, l].astype(jnp.float32),
                w_ref[pl.ds((gs * PLANE + l) * k_blk, k_blk), :],
                preferred_element_type=jnp.float32,
            )
            a = dz % 4
            accs[a] = p if accs[a] is None else accs[a] + p

        for dl in range(1, PLANE):
            rdmasB[dl - 1].wait_recv()
            ls = lax.rem(l + (PLANE - dl), PLANE)
            for gs in range(N_PLANES):
                p = jnp.dot(
                    xg_ref[gs, ls].astype(jnp.float32),
                    w_ref[pl.ds((gs * PLANE + ls) * k_blk, k_blk), :],
                    preferred_element_type=jnp.float32,
                )
                a = (dl * N_PLANES + gs) % 4
                accs[a] = p if accs[a] is None else accs[a] + p

        acc = (accs[0] + accs[1]) + (accs[2] + accs[3])
        out_ref[:, :] = jnp.maximum(acc, 0.0)

        for r in rdmasA:
            r.wait_send()
        for r in rdmasB:
            r.wait_send()

    return pl.pallas_call(
        body,
        out_shape=jax.ShapeDtypeStruct((m_blk, n_dim), jnp.float32),
        in_specs=[
            pl.BlockSpec(memory_space=pltpu.VMEM),
            pl.BlockSpec(memory_space=pltpu.VMEM),
        ],
        out_specs=pl.BlockSpec(memory_space=pltpu.VMEM),
        scratch_shapes=[
            pltpu.VMEM((N_DEV, m_blk, k_blk), jnp.bfloat16),
            pltpu.VMEM((N_PLANES, PLANE, m_blk, k_blk), jnp.bfloat16),
            pltpu.VMEM((N_PLANES, PLANE, m_blk, k_blk), jnp.bfloat16),
            pltpu.SemaphoreType.DMA((N_PLANES - 1,)),
            pltpu.SemaphoreType.DMA((N_PLANES - 1,)),
            pltpu.SemaphoreType.DMA((PLANE - 1,)),
            pltpu.SemaphoreType.DMA((PLANE - 1,)),
            pltpu.SemaphoreType.REGULAR,
            pltpu.SemaphoreType.DMA,
        ],
        compiler_params=pltpu.CompilerParams(
            collective_id=0,
            vmem_limit_bytes=100 * 1024 * 1024,
        ),
    )(x, w_mat)


# device time: 38063 ns/iter; 1.1477x vs baseline; 1.1477x over previous
import os

import jax
import jax.numpy as jnp
from jax import lax
from jax.experimental import pallas as pl
from jax.experimental.pallas import tpu as pltpu

N_DEV = 32
N_GROUPS = 4
GROUP = 8
_SKIP_DOTS = os.environ.get("KERNEL_SKIP_DOTS") == "1"


def _group_sizes():
    sizes = []
    rem = N_DEV - 1
    for _ in range(N_GROUPS):
        sizes.append(min(GROUP, rem))
        rem -= sizes[-1]
    return sizes


def kernel(x, w_mat):
    m_total, k_blk = x.shape
    k_total, n_dim = w_mat.shape
    m_blk = m_total // N_DEV
    sizes = _group_sizes()

    def body(x_ref, w_ref, out_ref, xs_ref, xg_ref,
             send_sems, recv_sems, round_sems):
        me = lax.axis_index("i")

        xs_ref[:, :, :] = (
            x_ref[:, :].astype(jnp.bfloat16).reshape(N_DEV, m_blk, k_blk)
        )

        def send_to(d):
            tgt = lax.rem(me + d, N_DEV)
            grp = (d - 1) // GROUP
            rdma = pltpu.make_async_remote_copy(
                src_ref=xs_ref.at[tgt],
                dst_ref=xg_ref.at[d - 1],
                send_sem=send_sems.at[grp],
                recv_sem=recv_sems.at[grp],
                device_id=(tgt,),
                device_id_type=pl.DeviceIdType.MESH,
            )
            rdma.start()

        barrier_sem = pltpu.get_barrier_semaphore()

        def round_sig(k):
            peer = lax.rem(me + (N_DEV - 2**k), N_DEV)
            sem = barrier_sem if k == 0 else round_sems.at[k - 1]
            pl.semaphore_signal(
                sem, inc=1,
                device_id=(peer,), device_id_type=pl.DeviceIdType.MESH,
            )

        round_sig(0)
        for k in range(5):
            pl.semaphore_wait(
                barrier_sem if k == 0 else round_sems.at[k - 1], 1
            )
            if k < 4:
                round_sig(k + 1)
            for d in range(2**k, min(2**(k + 1), N_DEV)):
                send_to(d)

        accs = [None, None, None, None]
        accs[0] = jnp.dot(
            xs_ref[me].astype(jnp.float32),
            w_ref[pl.ds(me * k_blk, k_blk), :],
            preferred_element_type=jnp.float32,
        )

        group_waits = []
        base = 0
        for j, sz in enumerate(sizes):
            group_waits.append(pltpu.make_async_remote_copy(
                src_ref=xs_ref.at[pl.ds(0, sz)],
                dst_ref=xg_ref.at[pl.ds(base, sz)],
                send_sem=send_sems.at[j],
                recv_sem=recv_sems.at[j],
                device_id=(0,),
                device_id_type=pl.DeviceIdType.MESH,
            ))
            base += sz

        base = 0
        for j, sz in enumerate(sizes):
            group_waits[j].wait_recv()
            if not _SKIP_DOTS:
                for slot in range(base, base + sz):
                    src_dev = lax.rem(me + (N_DEV - 1 - slot), N_DEV)
                    p = jnp.dot(
                        xg_ref[slot].astype(jnp.float32),
                        w_ref[pl.ds(src_dev * k_blk, k_blk), :],
                        preferred_element_type=jnp.float32,
                    )
                    a = (slot + 1) % 4
                    accs[a] = p if accs[a] is None else accs[a] + p
            base += sz

        if _SKIP_DOTS:
            acc = accs[0] + xg_ref[0].astype(jnp.float32)[0:1, 0:1]
        else:
            acc = (accs[0] + accs[1]) + (accs[2] + accs[3])
        out_ref[:, :] = jnp.maximum(acc, 0.0)

        for j in range(N_GROUPS):
            group_waits[j].wait_send()

    return pl.pallas_call(
        body,
        out_shape=jax.ShapeDtypeStruct((m_blk, n_dim), jnp.float32),
        in_specs=[
            pl.BlockSpec(memory_space=pltpu.VMEM),
            pl.BlockSpec(memory_space=pltpu.VMEM),
        ],
        out_specs=pl.BlockSpec(memory_space=pltpu.VMEM),
        scratch_shapes=[
            pltpu.VMEM((N_DEV, m_blk, k_blk), jnp.bfloat16),
            pltpu.VMEM((N_DEV - 1, m_blk, k_blk), jnp.bfloat16),
            pltpu.SemaphoreType.DMA((N_GROUPS,)),
            pltpu.SemaphoreType.DMA((N_GROUPS,)),
            pltpu.SemaphoreType.REGULAR((4,)),
        ],
        compiler_params=pltpu.CompilerParams(
            collective_id=0,
            vmem_limit_bytes=100 * 1024 * 1024,
        ),
    )(x, w_mat)


# device time: 38042 ns/iter; 1.1483x vs baseline; 1.0006x over previous
import os

import jax
import jax.numpy as jnp
from jax import lax
from jax.experimental import pallas as pl
from jax.experimental.pallas import tpu as pltpu

N_DEV = 32
N_GROUPS = 4
GROUP = 8
_SKIP_DOTS = os.environ.get("KERNEL_SKIP_DOTS") == "1"


def _group_sizes():
    sizes = []
    rem = N_DEV - 1
    for _ in range(N_GROUPS):
        sizes.append(min(GROUP, rem))
        rem -= sizes[-1]
    return sizes


def kernel(x, w_mat):
    m_total, k_blk = x.shape
    k_total, n_dim = w_mat.shape
    m_blk = m_total // N_DEV
    sizes = _group_sizes()

    def body(x_ref, w_ref, out_ref, xs_ref, xg_ref,
             send_sems, recv_sems, round_sems):
        me = lax.axis_index("i")

        def send_to(d):
            tgt = lax.rem(me + d, N_DEV)
            grp = (d - 1) // GROUP
            rdma = pltpu.make_async_remote_copy(
                src_ref=xs_ref.at[tgt],
                dst_ref=xg_ref.at[d - 1],
                send_sem=send_sems.at[grp],
                recv_sem=recv_sems.at[grp],
                device_id=(tgt,),
                device_id_type=pl.DeviceIdType.MESH,
            )
            rdma.start()

        barrier_sem = pltpu.get_barrier_semaphore()

        def round_sig(k):
            peer = lax.rem(me + (N_DEV - 2**k), N_DEV)
            sem = barrier_sem if k == 0 else round_sems.at[k - 1]
            pl.semaphore_signal(
                sem, inc=1,
                device_id=(peer,), device_id_type=pl.DeviceIdType.MESH,
            )

        round_sig(0)

        xs_ref[:, :, :] = (
            x_ref[:, :].astype(jnp.bfloat16).reshape(N_DEV, m_blk, k_blk)
        )

        for k in range(5):
            pl.semaphore_wait(
                barrier_sem if k == 0 else round_sems.at[k - 1], 1
            )
            if k < 4:
                round_sig(k + 1)
            for d in range(2**k, min(2**(k + 1), N_DEV)):
                send_to(d)

        accs = [None, None, None, None]
        accs[0] = jnp.dot(
            xs_ref[me].astype(jnp.float32),
            w_ref[pl.ds(me * k_blk, k_blk), :],
            preferred_element_type=jnp.float32,
        )

        group_waits = []
        base = 0
        for j, sz in enumerate(sizes):
            group_waits.append(pltpu.make_async_remote_copy(
                src_ref=xs_ref.at[pl.ds(0, sz)],
                dst_ref=xg_ref.at[pl.ds(base, sz)],
                send_sem=send_sems.at[j],
                recv_sem=recv_sems.at[j],
                device_id=(0,),
                device_id_type=pl.DeviceIdType.MESH,
            ))
            base += sz

        base = 0
        for j, sz in enumerate(sizes):
            group_waits[j].wait_recv()
            if not _SKIP_DOTS:
                for slot in range(base, base + sz):
                    src_dev = lax.rem(me + (N_DEV - 1 - slot), N_DEV)
                    p = jnp.dot(
                        xg_ref[slot].astype(jnp.float32),
                        w_ref[pl.ds(src_dev * k_blk, k_blk), :],
                        preferred_element_type=jnp.float32,
                    )
                    a = (slot + 1) % 4
                    accs[a] = p if accs[a] is None else accs[a] + p
            base += sz

        if _SKIP_DOTS:
            acc = accs[0] + xg_ref[0].astype(jnp.float32)[0:1, 0:1]
        else:
            acc = (accs[0] + accs[1]) + (accs[2] + accs[3])
        out_ref[:, :] = jnp.maximum(acc, 0.0)

        for j in range(N_GROUPS):
            group_waits[j].wait_send()

    return pl.pallas_call(
        body,
        out_shape=jax.ShapeDtypeStruct((m_blk, n_dim), jnp.float32),
        in_specs=[
            pl.BlockSpec(memory_space=pltpu.VMEM),
            pl.BlockSpec(memory_space=pltpu.VMEM),
        ],
        out_specs=pl.BlockSpec(memory_space=pltpu.VMEM),
        scratch_shapes=[
            pltpu.VMEM((N_DEV, m_blk, k_blk), jnp.bfloat16),
            pltpu.VMEM((N_DEV - 1, m_blk, k_blk), jnp.bfloat16),
            pltpu.SemaphoreType.DMA((N_GROUPS,)),
            pltpu.SemaphoreType.DMA((N_GROUPS,)),
            pltpu.SemaphoreType.REGULAR((4,)),
        ],
        compiler_params=pltpu.CompilerParams(
            collective_id=0,
            vmem_limit_bytes=100 * 1024 * 1024,
        ),
    )(x, w_mat)


# device time: 37771 ns/iter; 1.1565x vs baseline; 1.0072x over previous
import os

import jax
import jax.numpy as jnp
from jax import lax
from jax.experimental import pallas as pl
from jax.experimental.pallas import tpu as pltpu

N_DEV = 32
N_GROUPS = 8
GROUP = 4
_SKIP_DOTS = os.environ.get("KERNEL_SKIP_DOTS") == "1"


def _group_sizes():
    sizes = []
    rem = N_DEV - 1
    for _ in range(N_GROUPS):
        sizes.append(min(GROUP, rem))
        rem -= sizes[-1]
    return sizes


def kernel(x, w_mat):
    m_total, k_blk = x.shape
    k_total, n_dim = w_mat.shape
    m_blk = m_total // N_DEV
    sizes = _group_sizes()

    def body(x_ref, w_ref, out_ref, xs_ref, xg_ref,
             send_sems, recv_sems, round_sems):
        me = lax.axis_index("i")

        def send_to(d):
            tgt = lax.rem(me + d, N_DEV)
            grp = (d - 1) // GROUP
            rdma = pltpu.make_async_remote_copy(
                src_ref=xs_ref.at[tgt],
                dst_ref=xg_ref.at[d - 1],
                send_sem=send_sems.at[grp],
                recv_sem=recv_sems.at[grp],
                device_id=(tgt,),
                device_id_type=pl.DeviceIdType.MESH,
            )
            rdma.start()

        barrier_sem = pltpu.get_barrier_semaphore()

        def round_sig(k):
            peer = lax.rem(me + (N_DEV - 2**k), N_DEV)
            sem = barrier_sem if k == 0 else round_sems.at[k - 1]
            pl.semaphore_signal(
                sem, inc=1,
                device_id=(peer,), device_id_type=pl.DeviceIdType.MESH,
            )

        round_sig(0)

        xs_ref[:, :, :] = (
            x_ref[:, :].astype(jnp.bfloat16).reshape(N_DEV, m_blk, k_blk)
        )

        for k in range(5):
            pl.semaphore_wait(
                barrier_sem if k == 0 else round_sems.at[k - 1], 1
            )
            if k < 4:
                round_sig(k + 1)
            for d in range(2**k, min(2**(k + 1), N_DEV)):
                send_to(d)

        accs = [None, None, None, None]
        accs[0] = jnp.dot(
            xs_ref[me].astype(jnp.float32),
            w_ref[pl.ds(me * k_blk, k_blk), :],
            preferred_element_type=jnp.float32,
        )

        group_waits = []
        base = 0
        for j, sz in enumerate(sizes):
            group_waits.append(pltpu.make_async_remote_copy(
                src_ref=xs_ref.at[pl.ds(0, sz)],
                dst_ref=xg_ref.at[pl.ds(base, sz)],
                send_sem=send_sems.at[j],
                recv_sem=recv_sems.at[j],
                device_id=(0,),
                device_id_type=pl.DeviceIdType.MESH,
            ))
            base += sz

        base = 0
        for j, sz in enumerate(sizes):
            group_waits[j].wait_recv()
            if not _SKIP_DOTS:
                for slot in range(base, base + sz):
                    src_dev = lax.rem(me + (N_DEV - 1 - slot), N_DEV)
                    p = jnp.dot(
                        xg_ref[slot].astype(jnp.float32),
                        w_ref[pl.ds(src_dev * k_blk, k_blk), :],
                        preferred_element_type=jnp.float32,
                    )
                    a = (slot + 1) % 4
                    accs[a] = p if accs[a] is None else accs[a] + p
            base += sz

        if _SKIP_DOTS:
            acc = accs[0] + xg_ref[0].astype(jnp.float32)[0:1, 0:1]
        else:
            acc = (accs[0] + accs[1]) + (accs[2] + accs[3])
        out_ref[:, :] = jnp.maximum(acc, 0.0)

        for j in range(N_GROUPS):
            group_waits[j].wait_send()

    return pl.pallas_call(
        body,
        out_shape=jax.ShapeDtypeStruct((m_blk, n_dim), jnp.float32),
        in_specs=[
            pl.BlockSpec(memory_space=pltpu.VMEM),
            pl.BlockSpec(memory_space=pltpu.VMEM),
        ],
        out_specs=pl.BlockSpec(memory_space=pltpu.VMEM),
        scratch_shapes=[
            pltpu.VMEM((N_DEV, m_blk, k_blk), jnp.bfloat16),
            pltpu.VMEM((N_DEV - 1, m_blk, k_blk), jnp.bfloat16),
            pltpu.SemaphoreType.DMA((N_GROUPS,)),
            pltpu.SemaphoreType.DMA((N_GROUPS,)),
            pltpu.SemaphoreType.REGULAR((4,)),
        ],
        compiler_params=pltpu.CompilerParams(
            collective_id=0,
            vmem_limit_bytes=100 * 1024 * 1024,
        ),
    )(x, w_mat)


# device time: 37584 ns/iter; 1.1623x vs baseline; 1.0050x over previous
import jax
import jax.numpy as jnp
from jax import lax
from jax.experimental import pallas as pl
from jax.experimental.pallas import tpu as pltpu

N_DEV = 32
N_GROUPS = 8
GROUP = 4


def _group_sizes():
    sizes = []
    rem = N_DEV - 1
    for _ in range(N_GROUPS):
        sizes.append(min(GROUP, rem))
        rem -= sizes[-1]
    return sizes


def kernel(x, w_mat):
    m_total, k_blk = x.shape
    k_total, n_dim = w_mat.shape
    m_blk = m_total // N_DEV
    sizes = _group_sizes()

    def body(x_ref, w_ref, out_ref, xs_ref, xg_ref,
             send_sems, recv_sems, round_sems):
        me = lax.axis_index("i")

        def send_to(d):
            tgt = lax.rem(me + d, N_DEV)
            grp = (d - 1) // GROUP
            rdma = pltpu.make_async_remote_copy(
                src_ref=xs_ref.at[tgt],
                dst_ref=xg_ref.at[d - 1],
                send_sem=send_sems.at[grp],
                recv_sem=recv_sems.at[grp],
                device_id=(tgt,),
                device_id_type=pl.DeviceIdType.MESH,
            )
            rdma.start()

        barrier_sem = pltpu.get_barrier_semaphore()

        def round_sig(k):
            peer = lax.rem(me + (N_DEV - 2**k), N_DEV)
            sem = barrier_sem if k == 0 else round_sems.at[k - 1]
            pl.semaphore_signal(
                sem, inc=1,
                device_id=(peer,), device_id_type=pl.DeviceIdType.MESH,
            )

        round_sig(0)

        xs_ref[:, :, :] = (
            x_ref[:, :].astype(jnp.bfloat16).reshape(N_DEV, m_blk, k_blk)
        )

        for k in range(5):
            pl.semaphore_wait(
                barrier_sem if k == 0 else round_sems.at[k - 1], 1
            )
            if k < 4:
                round_sig(k + 1)
            for d in range(2**k, min(2**(k + 1), N_DEV)):
                send_to(d)

        accs = [None, None, None, None]
        accs[0] = jnp.dot(
            xs_ref[me].astype(jnp.float32),
            w_ref[pl.ds(me * k_blk, k_blk), :],
            preferred_element_type=jnp.float32,
        )

        group_waits = []
        base = 0
        for j, sz in enumerate(sizes):
            group_waits.append(pltpu.make_async_remote_copy(
                src_ref=xs_ref.at[pl.ds(0, sz)],
                dst_ref=xg_ref.at[pl.ds(base, sz)],
                send_sem=send_sems.at[j],
                recv_sem=recv_sems.at[j],
                device_id=(0,),
                device_id_type=pl.DeviceIdType.MESH,
            ))
            base += sz

        base = 0
        for j, sz in enumerate(sizes):
            group_waits[j].wait_recv()
            for slot in range(base, base + sz):
                src_dev = lax.rem(me + (N_DEV - 1 - slot), N_DEV)
                p = jnp.dot(
                    xg_ref[slot].astype(jnp.float32),
                    w_ref[pl.ds(src_dev * k_blk, k_blk), :],
                    preferred_element_type=jnp.float32,
                )
                a = (slot + 1) % 4
                accs[a] = p if accs[a] is None else accs[a] + p
            base += sz

        acc = (accs[0] + accs[1]) + (accs[2] + accs[3])
        out_ref[:, :] = jnp.maximum(acc, 0.0)

        for j in range(N_GROUPS):
            group_waits[j].wait_send()

    return pl.pallas_call(
        body,
        out_shape=jax.ShapeDtypeStruct((m_blk, n_dim), jnp.float32),
        in_specs=[
            pl.BlockSpec(memory_space=pltpu.VMEM),
            pl.BlockSpec(memory_space=pltpu.VMEM),
        ],
        out_specs=pl.BlockSpec(memory_space=pltpu.VMEM),
        scratch_shapes=[
            pltpu.VMEM((N_DEV, m_blk, k_blk), jnp.bfloat16),
            pltpu.VMEM((N_DEV - 1, m_blk, k_blk), jnp.bfloat16),
            pltpu.SemaphoreType.DMA((N_GROUPS,)),
            pltpu.SemaphoreType.DMA((N_GROUPS,)),
            pltpu.SemaphoreType.REGULAR((4,)),
        ],
        compiler_params=pltpu.CompilerParams(
            collective_id=0,
            vmem_limit_bytes=100 * 1024 * 1024,
        ),
    )(x, w_mat)
